# baseline (speedup 1.0000x reference)
import functools

import jax
import jax.numpy as jnp
from jax import lax
from jax.experimental import pallas as pl
from jax.experimental.pallas import tpu as pltpu
from jax.experimental.pallas import tpu_sc as plsc

N_NODES = 10000
D = 128
N_EDGES = 320000

NC = 2
NS = 16
NW = NC * NS

E_PER_TILE = N_EDGES // NW
HCHUNK = E_PER_TILE // 16

K_EDGES = 80
NBLK = N_EDGES // (NW * K_EDGES)
NCHUNK = 5
CBLK = NBLK // NCHUNK
RPT = N_NODES // NS

_sc_mesh = plsc.VectorSubcoreMesh(core_axis_name="c", subcore_axis_name="s")


@functools.partial(
    pl.kernel,
    out_type=jax.ShapeDtypeStruct((NW, N_NODES), jnp.float32),
    mesh=_sc_mesh,
    scratch_types=[
        pltpu.VMEM((E_PER_TILE,), jnp.int32),
        pltpu.VMEM((N_NODES,), jnp.float32),
    ],
    compiler_params=pltpu.CompilerParams(needs_layout_passes=False),
)
def _sc_degree(dst_hbm, out_hbm, dst_v, hist_v):
    c = lax.axis_index("c")
    s = lax.axis_index("s")
    wid = s * NC + c

    zero = jnp.zeros((16,), jnp.float32)

    def zloop(i, carry):
        hist_v[pl.ds(i * 16, 16)] = zero
        return carry

    lax.fori_loop(0, HCHUNK, zloop, 0)

    pltpu.sync_copy(dst_hbm.at[wid], dst_v)

    one = jnp.ones((16,), jnp.float32)

    def hloop(i, carry):
        idx = dst_v[pl.ds(i * 16, 16)]
        plsc.addupdate_scatter(hist_v, [idx], one)
        return carry

    lax.fori_loop(0, HCHUNK, hloop, 0)

    pltpu.sync_copy(hist_v, out_hbm.at[wid])


@functools.partial(
    pl.kernel,
    out_type=jax.ShapeDtypeStruct((NC, NS, RPT, D), jnp.float32),
    mesh=_sc_mesh,
    scratch_types=[
        pltpu.VMEM((CBLK, K_EDGES), jnp.int32),
        pltpu.VMEM((CBLK, K_EDGES), jnp.int32),
        pltpu.VMEM((3, K_EDGES, D), jnp.float32),
        pltpu.VMEM_SHARED((N_NODES, D), jnp.float32),
        pltpu.SemaphoreType.DMA,
        pltpu.SemaphoreType.DMA,
    ],
)
def _sc_aggregate(g_hbm, src_hbm, dst_hbm, zeros_hbm, out_hbm,
                  idx_s, idx_d, rows, acc, gsem, ssem):
    c = lax.axis_index("c")
    s = lax.axis_index("s")
    wid = s * NC + c
    base = s * RPT

    pltpu.async_copy(zeros_hbm, acc.at[pl.ds(base, RPT)], gsem)
    pltpu.async_copy(src_hbm.at[wid, 0], idx_s, gsem)
    pltpu.async_copy(dst_hbm.at[wid, 0], idx_d, gsem)
    pltpu.make_async_copy(zeros_hbm, acc.at[pl.ds(base, RPT)], gsem).wait()
    pltpu.make_async_copy(src_hbm.at[wid, 0], idx_s, gsem).wait()
    pltpu.make_async_copy(dst_hbm.at[wid, 0], idx_d, gsem).wait()
    plsc.subcore_barrier()

    for cc in range(NCHUNK):
        if cc > 0:
            pltpu.async_copy(src_hbm.at[wid, cc], idx_s, gsem)
            pltpu.async_copy(dst_hbm.at[wid, cc], idx_d, gsem)
            pltpu.make_async_copy(src_hbm.at[wid, cc], idx_s, gsem).wait()
            pltpu.make_async_copy(dst_hbm.at[wid, cc], idx_d, gsem).wait()
        pltpu.async_copy(g_hbm.at[idx_s.at[0]], rows.at[0], gsem)
        pltpu.async_copy(g_hbm.at[idx_s.at[1]], rows.at[1], gsem)

        def blk(i, carry):
            buf = lax.rem(i, 3)
            pltpu.make_async_copy(g_hbm.at[idx_s.at[i]], rows.at[buf],
                                  gsem).wait()
            pltpu.async_copy(rows.at[buf], acc.at[idx_d.at[i]], ssem,
                             add=True)

            @pl.when(i > 0)
            def _():
                pbuf = lax.rem(i - 1, 3)
                pltpu.make_async_copy(rows.at[pbuf],
                                      acc.at[idx_d.at[i - 1]],
                                      ssem).wait()

            @pl.when(i + 2 < CBLK)
            def _():
                pltpu.async_copy(g_hbm.at[idx_s.at[i + 2]],
                                 rows.at[lax.rem(i + 2, 3)], gsem)

            return carry

        lax.fori_loop(0, CBLK, blk, 0)
        pltpu.make_async_copy(rows.at[(CBLK - 1) % 3],
                              acc.at[idx_d.at[CBLK - 1]], ssem).wait()

    plsc.subcore_barrier()
    pltpu.sync_copy(acc.at[pl.ds(base, RPT)], out_hbm.at[c, s])


BR = 1280
GRID = -(-N_NODES // BR)


def _mmh1_body(x_ref, w_ref, h_ref):
    h_ref[...] = jnp.dot(x_ref[...], w_ref[...],
                         preferred_element_type=jnp.float32)


def _tc_mmh1(x, W1):
    return pl.pallas_call(
        _mmh1_body,
        grid=(GRID,),
        in_specs=[
            pl.BlockSpec((BR, D), lambda i: (i, 0)),
            pl.BlockSpec((D, D), lambda i: (0, 0)),
        ],
        out_specs=pl.BlockSpec((BR, D), lambda i: (i, 0)),
        out_shape=jax.ShapeDtypeStruct((N_NODES, D), jnp.float32),
    )(x, W1)


def _scale1_body(h_ref, degp_ref, g_ref, dis_ref):
    deg = jnp.sum(degp_ref[...], axis=0) + 1.0
    dis2 = jnp.broadcast_to(lax.rsqrt(deg)[:, None], (BR, D))
    g_ref[...] = h_ref[...] * dis2
    dis_ref[...] = dis2


def _tc_scale1(h1, degp):
    return pl.pallas_call(
        _scale1_body,
        grid=(GRID,),
        in_specs=[
            pl.BlockSpec((BR, D), lambda i: (i, 0)),
            pl.BlockSpec((NW, BR), lambda i: (0, i)),
        ],
        out_specs=[
            pl.BlockSpec((BR, D), lambda i: (i, 0)),
            pl.BlockSpec((BR, D), lambda i: (i, 0)),
        ],
        out_shape=[
            jax.ShapeDtypeStruct((N_NODES, D), jnp.float32),
            jax.ShapeDtypeStruct((N_NODES, D), jnp.float32),
        ],
    )(h1, degp)


def _ep1mm2_body(s1_ref, g1_ref, dis_ref, b1_ref, w_ref, g2_ref):
    dis = dis_ref[...]
    z1 = (s1_ref[0] + s1_ref[1] + g1_ref[...]) * dis + b1_ref[...]
    a1 = jnp.maximum(z1, 0.0)
    h2 = jnp.dot(a1, w_ref[...], preferred_element_type=jnp.float32)
    g2_ref[...] = h2 * dis


def _tc_ep1mm2(s1, g1, dis2, b1, W2):
    return pl.pallas_call(
        _ep1mm2_body,
        grid=(GRID,),
        in_specs=[
            pl.BlockSpec((NC, BR, D), lambda i: (0, i, 0)),
            pl.BlockSpec((BR, D), lambda i: (i, 0)),
            pl.BlockSpec((BR, D), lambda i: (i, 0)),
            pl.BlockSpec((1, D), lambda i: (0, 0)),
            pl.BlockSpec((D, D), lambda i: (0, 0)),
        ],
        out_specs=pl.BlockSpec((BR, D), lambda i: (i, 0)),
        out_shape=jax.ShapeDtypeStruct((N_NODES, D), jnp.float32),
    )(s1, g1, dis2, b1, W2)


def _ep2_body(s2_ref, g2_ref, dis_ref, b2_ref, out_ref):
    out_ref[...] = ((s2_ref[0] + s2_ref[1] + g2_ref[...]) * dis_ref[...]
                    + b2_ref[...])


def _tc_ep2(s2, g2, dis2, b2):
    return pl.pallas_call(
        _ep2_body,
        grid=(GRID,),
        in_specs=[
            pl.BlockSpec((NC, BR, D), lambda i: (0, i, 0)),
            pl.BlockSpec((BR, D), lambda i: (i, 0)),
            pl.BlockSpec((BR, D), lambda i: (i, 0)),
            pl.BlockSpec((1, D), lambda i: (0, 0)),
        ],
        out_specs=pl.BlockSpec((BR, D), lambda i: (i, 0)),
        out_shape=jax.ShapeDtypeStruct((N_NODES, D), jnp.float32),
    )(s2, g2, dis2, b2)


def kernel(x, edge_index, W1, b1, W2, b2):
    src = edge_index[0].astype(jnp.int32).reshape(NW, NCHUNK, CBLK, K_EDGES)
    dst = edge_index[1].astype(jnp.int32).reshape(NW, NCHUNK, CBLK, K_EDGES)
    dst_t = edge_index[1].astype(jnp.int32).reshape(NW, E_PER_TILE)
    zeros = jnp.zeros((RPT, D), jnp.float32)

    degp = _sc_degree(dst_t)
    h1 = _tc_mmh1(x, W1)
    g1, dis2 = _tc_scale1(h1, degp)
    s1 = _sc_aggregate(g1, src, dst, zeros).reshape(NC, N_NODES, D)
    g2 = _tc_ep1mm2(s1, g1, dis2, b1.reshape(1, D), W2)
    s2 = _sc_aggregate(g2, src, dst, zeros).reshape(NC, N_NODES, D)
    out = _tc_ep2(s2, g2, dis2, b2.reshape(1, D))
    return out


# speedup vs baseline: 1.0073x; 1.0073x over previous
import functools

import jax
import jax.numpy as jnp
from jax import lax
from jax.experimental import pallas as pl
from jax.experimental.pallas import tpu as pltpu
from jax.experimental.pallas import tpu_sc as plsc

N_NODES = 10000
D = 128
N_EDGES = 320000

NC = 2
NS = 16
NW = NC * NS

E_PER_TILE = N_EDGES // NW
HCHUNK = E_PER_TILE // 16

K_EDGES = 80
NBLK = N_EDGES // (NW * K_EDGES)
NCHUNK = 5
CBLK = NBLK // NCHUNK
RPT = N_NODES // NS

_sc_mesh = plsc.VectorSubcoreMesh(core_axis_name="c", subcore_axis_name="s")


@functools.partial(
    pl.kernel,
    out_type=jax.ShapeDtypeStruct((NW, N_NODES), jnp.float32),
    mesh=_sc_mesh,
    scratch_types=[
        pltpu.VMEM((E_PER_TILE,), jnp.int32),
        pltpu.VMEM((N_NODES,), jnp.float32),
    ],
    compiler_params=pltpu.CompilerParams(needs_layout_passes=False),
)
def _sc_degree(dst_hbm, out_hbm, dst_v, hist_v):
    c = lax.axis_index("c")
    s = lax.axis_index("s")
    wid = s * NC + c

    zero = jnp.zeros((16,), jnp.float32)

    def zloop(i, carry):
        hist_v[pl.ds(i * 16, 16)] = zero
        return carry

    lax.fori_loop(0, HCHUNK, zloop, 0)

    pltpu.sync_copy(dst_hbm.at[wid], dst_v)

    one = jnp.ones((16,), jnp.float32)

    def hloop(i, carry):
        idx = dst_v[pl.ds(i * 16, 16)]
        plsc.addupdate_scatter(hist_v, [idx], one)
        return carry

    lax.fori_loop(0, HCHUNK, hloop, 0)

    pltpu.sync_copy(hist_v, out_hbm.at[wid])


@functools.partial(
    pl.kernel,
    out_type=jax.ShapeDtypeStruct((NC, NS, RPT, D), jnp.float32),
    mesh=_sc_mesh,
    scratch_types=[
        pltpu.VMEM((CBLK, K_EDGES), jnp.int32),
        pltpu.VMEM((CBLK, K_EDGES), jnp.int32),
        pltpu.VMEM((3, K_EDGES, D), jnp.float32),
        pltpu.VMEM_SHARED((N_NODES, D), jnp.float32),
        pltpu.SemaphoreType.DMA,
        pltpu.SemaphoreType.DMA,
    ],
)
def _sc_aggregate(g_hbm, src_hbm, dst_hbm, zeros_hbm, out_hbm,
                  idx_s, idx_d, rows, acc, gsem, ssem):
    c = lax.axis_index("c")
    s = lax.axis_index("s")
    wid = s * NC + c
    base = s * RPT

    pltpu.async_copy(zeros_hbm, acc.at[pl.ds(base, RPT)], gsem)
    pltpu.async_copy(src_hbm.at[wid, 0], idx_s, gsem)
    pltpu.async_copy(dst_hbm.at[wid, 0], idx_d, gsem)
    pltpu.make_async_copy(zeros_hbm, acc.at[pl.ds(base, RPT)], gsem).wait()
    pltpu.make_async_copy(src_hbm.at[wid, 0], idx_s, gsem).wait()
    pltpu.make_async_copy(dst_hbm.at[wid, 0], idx_d, gsem).wait()
    plsc.subcore_barrier()

    for cc in range(NCHUNK):
        if cc > 0:
            pltpu.async_copy(src_hbm.at[wid, cc], idx_s, gsem)
            pltpu.async_copy(dst_hbm.at[wid, cc], idx_d, gsem)
            pltpu.make_async_copy(src_hbm.at[wid, cc], idx_s, gsem).wait()
            pltpu.make_async_copy(dst_hbm.at[wid, cc], idx_d, gsem).wait()
        pltpu.async_copy(g_hbm.at[idx_s.at[0]], rows.at[0], gsem)
        pltpu.async_copy(g_hbm.at[idx_s.at[1]], rows.at[1], gsem)

        def blk(i, carry):
            buf = lax.rem(i, 3)
            pltpu.make_async_copy(g_hbm.at[idx_s.at[i]], rows.at[buf],
                                  gsem).wait()
            pltpu.async_copy(rows.at[buf], acc.at[idx_d.at[i]], ssem,
                             add=True)

            @pl.when(i > 0)
            def _():
                pbuf = lax.rem(i - 1, 3)
                pltpu.make_async_copy(rows.at[pbuf],
                                      acc.at[idx_d.at[i - 1]],
                                      ssem).wait()

            @pl.when(i + 2 < CBLK)
            def _():
                pltpu.async_copy(g_hbm.at[idx_s.at[i + 2]],
                                 rows.at[lax.rem(i + 2, 3)], gsem)

            return carry

        lax.fori_loop(0, CBLK, blk, 0)
        pltpu.make_async_copy(rows.at[(CBLK - 1) % 3],
                              acc.at[idx_d.at[CBLK - 1]], ssem).wait()

    plsc.subcore_barrier()
    pltpu.sync_copy(acc.at[pl.ds(base, RPT)], out_hbm.at[c, s])


BR = 1280
GRID = -(-N_NODES // BR)


def _mm1_body(x_ref, w_ref, degp_ref, g_ref, dis_ref):
    deg = jnp.sum(degp_ref[...], axis=0) + 1.0
    dis2 = jnp.broadcast_to(lax.rsqrt(deg)[:, None], (BR, D))
    h = jnp.dot(x_ref[...], w_ref[...], preferred_element_type=jnp.float32)
    g_ref[...] = h * dis2
    dis_ref[...] = dis2


def _tc_mm1(x, W1, degp):
    return pl.pallas_call(
        _mm1_body,
        grid=(GRID,),
        in_specs=[
            pl.BlockSpec((BR, D), lambda i: (i, 0)),
            pl.BlockSpec((D, D), lambda i: (0, 0)),
            pl.BlockSpec((NW, BR), lambda i: (0, i)),
        ],
        out_specs=[
            pl.BlockSpec((BR, D), lambda i: (i, 0)),
            pl.BlockSpec((BR, D), lambda i: (i, 0)),
        ],
        out_shape=[
            jax.ShapeDtypeStruct((N_NODES, D), jnp.float32),
            jax.ShapeDtypeStruct((N_NODES, D), jnp.float32),
        ],
    )(x, W1, degp)


def _ep1mm2_body(s1_ref, g1_ref, dis_ref, b1_ref, w_ref, g2_ref):
    dis = dis_ref[...]
    z1 = (s1_ref[0] + s1_ref[1] + g1_ref[...]) * dis + b1_ref[...]
    a1 = jnp.maximum(z1, 0.0)
    h2 = jnp.dot(a1, w_ref[...], preferred_element_type=jnp.float32)
    g2_ref[...] = h2 * dis


def _tc_ep1mm2(s1, g1, dis2, b1, W2):
    return pl.pallas_call(
        _ep1mm2_body,
        grid=(GRID,),
        in_specs=[
            pl.BlockSpec((NC, BR, D), lambda i: (0, i, 0)),
            pl.BlockSpec((BR, D), lambda i: (i, 0)),
            pl.BlockSpec((BR, D), lambda i: (i, 0)),
            pl.BlockSpec((1, D), lambda i: (0, 0)),
            pl.BlockSpec((D, D), lambda i: (0, 0)),
        ],
        out_specs=pl.BlockSpec((BR, D), lambda i: (i, 0)),
        out_shape=jax.ShapeDtypeStruct((N_NODES, D), jnp.float32),
    )(s1, g1, dis2, b1, W2)


def _ep2_body(s2_ref, g2_ref, dis_ref, b2_ref, out_ref):
    out_ref[...] = ((s2_ref[0] + s2_ref[1] + g2_ref[...]) * dis_ref[...]
                    + b2_ref[...])


def _tc_ep2(s2, g2, dis2, b2):
    return pl.pallas_call(
        _ep2_body,
        grid=(GRID,),
        in_specs=[
            pl.BlockSpec((NC, BR, D), lambda i: (0, i, 0)),
            pl.BlockSpec((BR, D), lambda i: (i, 0)),
            pl.BlockSpec((BR, D), lambda i: (i, 0)),
            pl.BlockSpec((1, D), lambda i: (0, 0)),
        ],
        out_specs=pl.BlockSpec((BR, D), lambda i: (i, 0)),
        out_shape=jax.ShapeDtypeStruct((N_NODES, D), jnp.float32),
    )(s2, g2, dis2, b2)


def kernel(x, edge_index, W1, b1, W2, b2):
    src = edge_index[0].astype(jnp.int32).reshape(NW, NCHUNK, CBLK, K_EDGES)
    dst = edge_index[1].astype(jnp.int32).reshape(NW, NCHUNK, CBLK, K_EDGES)
    dst_t = edge_index[1].astype(jnp.int32).reshape(NW, E_PER_TILE)
    zeros = jnp.zeros((RPT, D), jnp.float32)

    degp = _sc_degree(dst_t)
    g1, dis2 = _tc_mm1(x, W1, degp)
    s1 = _sc_aggregate(g1, src, dst, zeros).reshape(NC, N_NODES, D)
    g2 = _tc_ep1mm2(s1, g1, dis2, b1.reshape(1, D), W2)
    s2 = _sc_aggregate(g2, src, dst, zeros).reshape(NC, N_NODES, D)
    out = _tc_ep2(s2, g2, dis2, b2.reshape(1, D))
    return out


# speedup vs baseline: 1.0202x; 1.0128x over previous
import functools

import jax
import jax.numpy as jnp
from jax import lax
from jax.experimental import pallas as pl
from jax.experimental.pallas import tpu as pltpu
from jax.experimental.pallas import tpu_sc as plsc

N_NODES = 10000
D = 128
N_EDGES = 320000

NC = 2
NS = 16
NW = NC * NS

E_PER_TILE = N_EDGES // NW
HCHUNK = E_PER_TILE // 16

K_EDGES = 80
NBLK = N_EDGES // (NW * K_EDGES)
NCHUNK = 5
CBLK = NBLK // NCHUNK
RPT = N_NODES // NS

_sc_mesh = plsc.VectorSubcoreMesh(core_axis_name="c", subcore_axis_name="s")


@functools.partial(
    pl.kernel,
    out_type=jax.ShapeDtypeStruct((NW, N_NODES), jnp.float32),
    mesh=_sc_mesh,
    scratch_types=[
        pltpu.VMEM((E_PER_TILE,), jnp.int32),
        pltpu.VMEM((N_NODES,), jnp.float32),
    ],
    compiler_params=pltpu.CompilerParams(needs_layout_passes=False),
)
def _sc_degree(dst_hbm, out_hbm, dst_v, hist_v):
    c = lax.axis_index("c")
    s = lax.axis_index("s")
    wid = s * NC + c

    zero = jnp.zeros((16,), jnp.float32)

    def zloop(i, carry):
        hist_v[pl.ds(i * 16, 16)] = zero
        return carry

    lax.fori_loop(0, HCHUNK, zloop, 0)

    pltpu.sync_copy(dst_hbm.at[wid], dst_v)

    one = jnp.ones((16,), jnp.float32)

    def hloop(i, carry):
        idx = dst_v[pl.ds(i * 16, 16)]
        plsc.addupdate_scatter(hist_v, [idx], one)
        return carry

    lax.fori_loop(0, HCHUNK, hloop, 0)

    pltpu.sync_copy(hist_v, out_hbm.at[wid])


@functools.partial(
    pl.kernel,
    out_type=jax.ShapeDtypeStruct((NC, NS, RPT, D), jnp.float32),
    mesh=_sc_mesh,
    scratch_types=[
        pltpu.VMEM((2, CBLK, K_EDGES), jnp.int32),
        pltpu.VMEM((2, CBLK, K_EDGES), jnp.int32),
        pltpu.VMEM((3, K_EDGES, D), jnp.float32),
        pltpu.VMEM_SHARED((N_NODES, D), jnp.float32),
        pltpu.SemaphoreType.DMA,
        pltpu.SemaphoreType.DMA,
        pltpu.SemaphoreType.DMA,
    ],
)
def _sc_aggregate(g_hbm, src_hbm, dst_hbm, zeros_hbm, out_hbm,
                  idx_s, idx_d, rows, acc, gsem, ssem, isem):
    c = lax.axis_index("c")
    s = lax.axis_index("s")
    wid = s * NC + c
    base = s * RPT

    pltpu.async_copy(zeros_hbm, acc.at[pl.ds(base, RPT)], gsem)
    pltpu.async_copy(src_hbm.at[wid, 0], idx_s.at[0], gsem)
    pltpu.async_copy(dst_hbm.at[wid, 0], idx_d.at[0], gsem)
    pltpu.make_async_copy(zeros_hbm, acc.at[pl.ds(base, RPT)], gsem).wait()
    pltpu.make_async_copy(src_hbm.at[wid, 0], idx_s.at[0], gsem).wait()
    pltpu.make_async_copy(dst_hbm.at[wid, 0], idx_d.at[0], gsem).wait()
    plsc.subcore_barrier()

    for cc in range(NCHUNK):
        slot = cc % 2
        pltpu.async_copy(g_hbm.at[idx_s.at[slot, 0]], rows.at[0], gsem)
        pltpu.async_copy(g_hbm.at[idx_s.at[slot, 1]], rows.at[1], gsem)
        if cc + 1 < NCHUNK:
            pltpu.async_copy(src_hbm.at[wid, cc + 1],
                             idx_s.at[1 - slot], isem)
            pltpu.async_copy(dst_hbm.at[wid, cc + 1],
                             idx_d.at[1 - slot], isem)

        def blk(i, carry):
            buf = lax.rem(i, 3)
            pltpu.make_async_copy(g_hbm.at[idx_s.at[slot, i]],
                                  rows.at[buf], gsem).wait()
            pltpu.async_copy(rows.at[buf], acc.at[idx_d.at[slot, i]],
                             ssem, add=True)

            @pl.when(i > 0)
            def _():
                pbuf = lax.rem(i - 1, 3)
                pltpu.make_async_copy(rows.at[pbuf],
                                      acc.at[idx_d.at[slot, i - 1]],
                                      ssem).wait()

            @pl.when(i + 2 < CBLK)
            def _():
                pltpu.async_copy(g_hbm.at[idx_s.at[slot, i + 2]],
                                 rows.at[lax.rem(i + 2, 3)], gsem)

            return carry

        lax.fori_loop(0, CBLK, blk, 0)
        pltpu.make_async_copy(rows.at[(CBLK - 1) % 3],
                              acc.at[idx_d.at[slot, CBLK - 1]],
                              ssem).wait()
        if cc + 1 < NCHUNK:
            pltpu.make_async_copy(src_hbm.at[wid, cc + 1],
                                  idx_s.at[1 - slot], isem).wait()
            pltpu.make_async_copy(dst_hbm.at[wid, cc + 1],
                                  idx_d.at[1 - slot], isem).wait()

    plsc.subcore_barrier()
    pltpu.sync_copy(acc.at[pl.ds(base, RPT)], out_hbm.at[c, s])


BR = 1280
GRID = -(-N_NODES // BR)


def _mm1_body(x_ref, w_ref, degp_ref, g_ref, dis_ref):
    deg = jnp.sum(degp_ref[...], axis=0) + 1.0
    dis2 = jnp.broadcast_to(lax.rsqrt(deg)[:, None], (BR, D))
    h = jnp.dot(x_ref[...], w_ref[...], preferred_element_type=jnp.float32)
    g_ref[...] = h * dis2
    dis_ref[...] = dis2


def _tc_mm1(x, W1, degp):
    return pl.pallas_call(
        _mm1_body,
        grid=(GRID,),
        in_specs=[
            pl.BlockSpec((BR, D), lambda i: (i, 0)),
            pl.BlockSpec((D, D), lambda i: (0, 0)),
            pl.BlockSpec((NW, BR), lambda i: (0, i)),
        ],
        out_specs=[
            pl.BlockSpec((BR, D), lambda i: (i, 0)),
            pl.BlockSpec((BR, D), lambda i: (i, 0)),
        ],
        out_shape=[
            jax.ShapeDtypeStruct((N_NODES, D), jnp.float32),
            jax.ShapeDtypeStruct((N_NODES, D), jnp.float32),
        ],
    )(x, W1, degp)


def _ep1mm2_body(s1_ref, g1_ref, dis_ref, b1_ref, w_ref, g2_ref):
    dis = dis_ref[...]
    z1 = (s1_ref[0] + s1_ref[1] + g1_ref[...]) * dis + b1_ref[...]
    a1 = jnp.maximum(z1, 0.0)
    h2 = jnp.dot(a1, w_ref[...], preferred_element_type=jnp.float32)
    g2_ref[...] = h2 * dis


def _tc_ep1mm2(s1, g1, dis2, b1, W2):
    return pl.pallas_call(
        _ep1mm2_body,
        grid=(GRID,),
        in_specs=[
            pl.BlockSpec((NC, BR, D), lambda i: (0, i, 0)),
            pl.BlockSpec((BR, D), lambda i: (i, 0)),
            pl.BlockSpec((BR, D), lambda i: (i, 0)),
            pl.BlockSpec((1, D), lambda i: (0, 0)),
            pl.BlockSpec((D, D), lambda i: (0, 0)),
        ],
        out_specs=pl.BlockSpec((BR, D), lambda i: (i, 0)),
        out_shape=jax.ShapeDtypeStruct((N_NODES, D), jnp.float32),
    )(s1, g1, dis2, b1, W2)


def _ep2_body(s2_ref, g2_ref, dis_ref, b2_ref, out_ref):
    out_ref[...] = ((s2_ref[0] + s2_ref[1] + g2_ref[...]) * dis_ref[...]
                    + b2_ref[...])


def _tc_ep2(s2, g2, dis2, b2):
    return pl.pallas_call(
        _ep2_body,
        grid=(GRID,),
        in_specs=[
            pl.BlockSpec((NC, BR, D), lambda i: (0, i, 0)),
            pl.BlockSpec((BR, D), lambda i: (i, 0)),
            pl.BlockSpec((BR, D), lambda i: (i, 0)),
            pl.BlockSpec((1, D), lambda i: (0, 0)),
        ],
        out_specs=pl.BlockSpec((BR, D), lambda i: (i, 0)),
        out_shape=jax.ShapeDtypeStruct((N_NODES, D), jnp.float32),
    )(s2, g2, dis2, b2)


def kernel(x, edge_index, W1, b1, W2, b2):
    src = edge_index[0].astype(jnp.int32).reshape(NW, NCHUNK, CBLK, K_EDGES)
    dst = edge_index[1].astype(jnp.int32).reshape(NW, NCHUNK, CBLK, K_EDGES)
    dst_t = edge_index[1].astype(jnp.int32).reshape(NW, E_PER_TILE)
    zeros = jnp.zeros((RPT, D), jnp.float32)

    degp = _sc_degree(dst_t)
    g1, dis2 = _tc_mm1(x, W1, degp)
    s1 = _sc_aggregate(g1, src, dst, zeros).reshape(NC, N_NODES, D)
    g2 = _tc_ep1mm2(s1, g1, dis2, b1.reshape(1, D), W2)
    s2 = _sc_aggregate(g2, src, dst, zeros).reshape(NC, N_NODES, D)
    out = _tc_ep2(s2, g2, dis2, b2.reshape(1, D))
    return out
